# baseline (speedup 1.0000x reference)
import functools

import jax
import jax.numpy as jnp
from jax import lax
from jax.experimental import pallas as pl
from jax.experimental.pallas import tpu as pltpu
from jax.experimental.pallas import tpu_sc as plsc

_NC, _NS = 2, 16
_NW = _NC * _NS
_CH = 128


def _make_gather(n_idx: int, d: int):
    assert n_idx % (_NW * _CH) == 0
    per_w = n_idx // _NW
    nch = per_w // _CH

    mesh = plsc.VectorSubcoreMesh(core_axis_name="c", subcore_axis_name="s")

    @functools.partial(
        pl.kernel,
        out_type=jax.ShapeDtypeStruct((n_idx, d), jnp.float32),
        mesh=mesh,
        scratch_types=[
            pltpu.VMEM((nch, _CH), jnp.int32),
            pltpu.VMEM((_CH, d), jnp.float32),
            pltpu.SemaphoreType.DMA,
        ],
        compiler_params=pltpu.CompilerParams(use_tc_tiling_on_sc=False),
    )
    def gather(table_hbm, idx_hbm, out_hbm, idx_v, buf, sem):
        wid = lax.axis_index("s") * _NC + lax.axis_index("c")
        pltpu.sync_copy(idx_hbm.at[wid], idx_v)
        row0 = wid * per_w

        def step(j, carry):
            pltpu.async_copy(table_hbm.at[idx_v.at[j]], buf, sem).wait()
            pltpu.sync_copy(buf, out_hbm.at[pl.ds(row0 + j * _CH, _CH)])
            return carry

        lax.fori_loop(0, nch, step, 0)

    return gather


def kernel(input_tensor, indices):
    n_idx = indices.size
    d = input_tensor.shape[-1]
    idx = indices.astype(jnp.int32).reshape(_NW, n_idx // (_NW * _CH), _CH)
    out = _make_gather(n_idx, d)(input_tensor, idx)
    return out.reshape(indices.shape + (d,))


# speedup vs baseline: 1.0390x; 1.0390x over previous
import functools

import jax
import jax.numpy as jnp
from jax import lax
from jax.experimental import pallas as pl
from jax.experimental.pallas import tpu as pltpu
from jax.experimental.pallas import tpu_sc as plsc

_NC, _NS = 2, 16
_NW = _NC * _NS
_CH = 128
_SPB = 5
_BC = _CH * _SPB


def _make_gather(n_idx: int, d: int):
    per_w = n_idx // _NW
    nch = per_w // _CH
    nbc = per_w // _BC
    assert n_idx == _NW * nbc * _BC and nbc % 2 == 0 and nbc >= 4

    mesh = plsc.VectorSubcoreMesh(core_axis_name="c", subcore_axis_name="s")

    @functools.partial(
        pl.kernel,
        out_type=jax.ShapeDtypeStruct((n_idx, d), jnp.float32),
        mesh=mesh,
        scratch_types=[
            pltpu.VMEM((nch, _CH), jnp.int32),
            pltpu.VMEM((_BC, d), jnp.float32),
            pltpu.VMEM((_BC, d), jnp.float32),
            pltpu.SemaphoreType.DMA,
            pltpu.SemaphoreType.DMA,
            pltpu.SemaphoreType.DMA,
            pltpu.SemaphoreType.DMA,
        ],
        compiler_params=pltpu.CompilerParams(use_tc_tiling_on_sc=False),
    )
    def gather(table_hbm, idx_hbm, out_hbm, idx_v,
               buf_a, buf_b, gsem_a, gsem_b, wsem_a, wsem_b):
        wid = lax.axis_index("s") * _NC + lax.axis_index("c")
        pltpu.sync_copy(idx_hbm.at[wid], idx_v)
        row0 = wid * per_w

        def g_start(c, buf, sem):
            for k in range(_SPB):
                pltpu.make_async_copy(
                    table_hbm.at[idx_v.at[c * _SPB + k]],
                    buf.at[pl.ds(k * _CH, _CH)], sem).start()

        def g_wait(buf, sem):
            pltpu.make_async_copy(
                table_hbm.at[pl.ds(0, _BC)], buf, sem).wait()

        def w_start(c, buf, sem):
            pltpu.make_async_copy(
                buf, out_hbm.at[pl.ds(row0 + c * _BC, _BC)], sem).start()

        def w_wait(buf, sem):
            pltpu.make_async_copy(
                buf, out_hbm.at[pl.ds(row0, _BC)], sem).wait()

        g_start(0, buf_a, gsem_a)
        g_start(1, buf_b, gsem_b)

        def step(i, carry):
            c = 2 * i
            g_wait(buf_a, gsem_a)
            w_start(c, buf_a, wsem_a)
            g_wait(buf_b, gsem_b)
            w_start(c + 1, buf_b, wsem_b)
            w_wait(buf_a, wsem_a)
            g_start(c + 2, buf_a, gsem_a)
            w_wait(buf_b, wsem_b)
            g_start(c + 3, buf_b, gsem_b)
            return carry

        lax.fori_loop(0, nbc // 2 - 1, step, 0)

        c = nbc - 2
        g_wait(buf_a, gsem_a)
        w_start(c, buf_a, wsem_a)
        g_wait(buf_b, gsem_b)
        w_start(c + 1, buf_b, wsem_b)
        w_wait(buf_a, wsem_a)
        w_wait(buf_b, wsem_b)

    return gather


def kernel(input_tensor, indices):
    n_idx = indices.size
    d = input_tensor.shape[-1]
    idx = indices.astype(jnp.int32).reshape(_NW, n_idx // (_NW * _CH), _CH)
    out = _make_gather(n_idx, d)(input_tensor, idx)
    return out.reshape(indices.shape + (d,))
